# Initial kernel scaffold; baseline (speedup 1.0000x reference)
import jax
import jax.numpy as jnp
from jax.experimental import pallas as pl
from jax.experimental.pallas import tpu as pltpu

_BIG = 1e30


def _loss_kernel(w_ref, tgt_ref, pred_ref, out_ref):
    C, H, W, NB = pred_ref.shape

    tgt = tgt_ref[...]
    cls = jax.lax.broadcasted_iota(jnp.int32, (C, H, W, NB), 0)
    is_c = tgt[None, :, :, :] == cls
    m = jnp.where(is_c, jnp.float32(0.0), jnp.float32(_BIG))

    xio = jax.lax.broadcasted_iota(jnp.float32, (1, 1, W, 1), 2)

    def s1(i, g):
        slab = jax.lax.dynamic_slice_in_dim(m, i, 1, axis=2)
        dx = xio - i.astype(jnp.float32)
        return jnp.minimum(g, slab + dx * dx)

    g = jax.lax.fori_loop(
        0, W, s1, jnp.full((C, H, W, NB), _BIG, jnp.float32))

    yio = jax.lax.broadcasted_iota(jnp.float32, (1, H, 1, 1), 1)

    def s2(i, d):
        slab = jax.lax.dynamic_slice_in_dim(g, i, 1, axis=1)
        dy = yio - i.astype(jnp.float32)
        return jnp.minimum(d, slab + dy * dy)

    d2 = jax.lax.fori_loop(
        0, H, s2, jnp.full((C, H, W, NB), _BIG, jnp.float32))

    dist = jnp.sqrt(d2)

    logits = pred_ref[...]
    mx = jnp.max(logits, axis=0, keepdims=True)
    ex = jnp.exp(logits - mx)
    sm = ex * (1.0 / jnp.sum(ex, axis=0, keepdims=True))

    sdist = jnp.sum(sm * dist, axis=(1, 2))
    dmax = jnp.max(dist, axis=(1, 2))
    cmass = jnp.sum(jnp.where(is_c, sm, jnp.float32(0.0)), axis=(1, 2))

    per_c = sdist - dmax * cmass
    total = jnp.zeros((1, NB), jnp.float32)
    for c in range(C):
        total = total + w_ref[c] * per_c[c][None, :]
    out_ref[...] = total


def kernel(predictions, targets, weight):
    nb, nc, h, width = predictions.shape
    p = h * width

    if weight is None or len(weight) != nc:
        weight_arr = jnp.ones((nc,), jnp.float32)
    else:
        weight_arr = jnp.asarray(weight, jnp.float32)
    w_norm = (weight_arr / jnp.sum(weight_arr)).astype(jnp.float32)

    NB = 128
    num_blocks = nb // NB

    preds_t = jnp.transpose(predictions.astype(jnp.float32), (1, 2, 3, 0))
    tgts_t = jnp.transpose(targets.astype(jnp.int32), (1, 2, 0))

    grid_spec = pltpu.PrefetchScalarGridSpec(
        num_scalar_prefetch=0,
        grid=(num_blocks,),
        in_specs=[
            pl.BlockSpec(memory_space=pltpu.MemorySpace.SMEM),
            pl.BlockSpec((h, width, NB), lambda i: (0, 0, i)),
            pl.BlockSpec((nc, h, width, NB), lambda i: (0, 0, 0, i)),
        ],
        out_specs=pl.BlockSpec((1, NB), lambda i: (i, 0)),
    )

    partials = pl.pallas_call(
        _loss_kernel,
        out_shape=jax.ShapeDtypeStruct((num_blocks, NB), jnp.float32),
        grid_spec=grid_spec,
        compiler_params=pltpu.CompilerParams(
            dimension_semantics=("parallel",),
            vmem_limit_bytes=96 * 1024 * 1024),
    )(w_norm, tgts_t, preds_t)

    return jnp.sum(partials) / (nb * nc * p)


# speedup vs baseline: 20.4260x; 20.4260x over previous
import jax
import jax.numpy as jnp
from jax.experimental import pallas as pl
from jax.experimental.pallas import tpu as pltpu

_BIG = 1e30


def _loss_kernel(w_ref, tgt_ref, pred_ref, out_ref, m_ref, g_ref, d_ref):
    C, H, W, NB = pred_ref.shape

    tgt = tgt_ref[...]
    cls = jax.lax.broadcasted_iota(jnp.int32, (C, H, W, NB), 0)
    is_c = tgt[None, :, :, :] == cls
    m_ref[...] = jnp.where(is_c, jnp.float32(0.0), jnp.float32(_BIG))

    xio = jax.lax.broadcasted_iota(
        jnp.int32, (1, 1, W, 1), 2).astype(jnp.float32)
    g_ref[...] = jnp.full((C, H, W, NB), _BIG, jnp.float32)

    def s1(i, carry):
        dx = xio - i.astype(jnp.float32)
        slab = m_ref[:, :, pl.ds(i, 1), :]
        g_ref[...] = jnp.minimum(g_ref[...], slab + dx * dx)
        return carry

    jax.lax.fori_loop(0, W, s1, 0)

    yio = jax.lax.broadcasted_iota(
        jnp.int32, (1, H, 1, 1), 1).astype(jnp.float32)
    d_ref[...] = jnp.full((C, H, W, NB), _BIG, jnp.float32)

    def s2(i, carry):
        dy = yio - i.astype(jnp.float32)
        slab = g_ref[:, pl.ds(i, 1), :, :]
        d_ref[...] = jnp.minimum(d_ref[...], slab + dy * dy)
        return carry

    jax.lax.fori_loop(0, H, s2, 0)

    dist = jnp.sqrt(d_ref[...])

    logits = pred_ref[...]
    mx = jnp.max(logits, axis=0, keepdims=True)
    ex = jnp.exp(logits - mx)
    sm = ex * (1.0 / jnp.sum(ex, axis=0, keepdims=True))

    sdist = jnp.sum(sm * dist, axis=(1, 2))
    dmax = jnp.max(dist, axis=(1, 2))
    cmass = jnp.sum(jnp.where(is_c, sm, jnp.float32(0.0)), axis=(1, 2))

    per_c = sdist - dmax * cmass
    total = jnp.zeros((1, NB), jnp.float32)
    for c in range(C):
        total = total + w_ref[c] * per_c[c][None, :]
    out_ref[...] = total[None]


def kernel(predictions, targets, weight):
    nb, nc, h, width = predictions.shape
    p = h * width

    if weight is None or len(weight) != nc:
        weight_arr = jnp.ones((nc,), jnp.float32)
    else:
        weight_arr = jnp.asarray(weight, jnp.float32)
    w_norm = (weight_arr / jnp.sum(weight_arr)).astype(jnp.float32)

    NB = 128
    num_blocks = nb // NB

    preds_t = jnp.transpose(predictions.astype(jnp.float32), (1, 2, 3, 0))
    tgts_t = jnp.transpose(targets.astype(jnp.int32), (1, 2, 0))

    grid_spec = pltpu.PrefetchScalarGridSpec(
        num_scalar_prefetch=0,
        grid=(num_blocks,),
        in_specs=[
            pl.BlockSpec(memory_space=pltpu.MemorySpace.SMEM),
            pl.BlockSpec((h, width, NB), lambda i: (0, 0, i)),
            pl.BlockSpec((nc, h, width, NB), lambda i: (0, 0, 0, i)),
        ],
        out_specs=pl.BlockSpec((1, 1, NB), lambda i: (i, 0, 0)),
        scratch_shapes=[
            pltpu.VMEM((nc, h, width, NB), jnp.float32),
            pltpu.VMEM((nc, h, width, NB), jnp.float32),
            pltpu.VMEM((nc, h, width, NB), jnp.float32),
        ],
    )

    partials = pl.pallas_call(
        _loss_kernel,
        out_shape=jax.ShapeDtypeStruct((num_blocks, 1, NB), jnp.float32),
        grid_spec=grid_spec,
        compiler_params=pltpu.CompilerParams(
            dimension_semantics=("parallel",),
            vmem_limit_bytes=96 * 1024 * 1024),
    )(w_norm, tgts_t, preds_t)

    return jnp.sum(partials) / (nb * nc * p)
